# baseline (device time: 25839 ns/iter reference)
import jax
import jax.numpy as jnp
from jax import lax
from jax.experimental import pallas as pl
from jax.experimental.pallas import tpu as pltpu

N_DEV = 16
B, SQ, DMODEL = 2, 256, 512
HQ_LOCAL, DH = 4, 64
ROWS = B * SQ
HCHUNK = SQ // N_DEV
WINDOW = 128


def _body(x_ref, wq_ref, k_ref, v_ref, wo_ref, out_ref,
          part_ref, recv1_ref,
          send1, recv1, send2, recv2):
    my = lax.axis_index("i")

    barrier = pltpu.get_barrier_semaphore()
    for o in range(1, N_DEV):
        pl.semaphore_signal(
            barrier, inc=1,
            device_id=((my + o) % N_DEV,),
            device_id_type=pl.DeviceIdType.MESH,
        )

    qi = lax.broadcasted_iota(jnp.int32, (SQ, SQ), 0)
    ki = lax.broadcasted_iota(jnp.int32, (SQ, SQ), 1)
    mask = jnp.abs(qi - ki) <= WINDOW

    def p1_rdma(half, o):
        dst = (my + o) % N_DEV
        return pltpu.make_async_remote_copy(
            src_ref=part_ref.at[pl.ds(half * SQ + dst * HCHUNK, HCHUNK), :],
            dst_ref=recv1_ref.at[half * (N_DEV - 1) + o - 1],
            send_sem=send1.at[half * (N_DEV - 1) + o - 1],
            recv_sem=recv1.at[half * (N_DEV - 1) + o - 1],
            device_id=(dst,),
            device_id_type=pl.DeviceIdType.MESH,
        )

    def p2_rdma(half, o):
        dst = (my + o) % N_DEV
        return pltpu.make_async_remote_copy(
            src_ref=out_ref.at[pl.ds(half * SQ + my * HCHUNK, HCHUNK), :],
            dst_ref=out_ref.at[pl.ds(half * SQ + my * HCHUNK, HCHUNK), :],
            send_sem=send2.at[half * (N_DEV - 1) + o - 1],
            recv_sem=recv2.at[half * (N_DEV - 1) + o - 1],
            device_id=(dst,),
            device_id_type=pl.DeviceIdType.MESH,
        )

    wq = wq_ref[:, :]
    wo = wo_ref[:, :]
    for b in range(B):
        xb = x_ref[pl.ds(b * SQ, SQ), :]
        qb = lax.dot_general(xb, wq, (((1,), (0,)), ((), ())),
                             preferred_element_type=jnp.float32)
        qb = qb.astype(jnp.bfloat16)
        cparts = []
        for h in range(HQ_LOCAL):
            qbh = qb[:, h * DH:(h + 1) * DH]
            kbh = k_ref[b, h]
            vbh = v_ref[b, h]
            s = lax.dot_general(qbh, kbh, (((1,), (1,)), ((), ())),
                                preferred_element_type=jnp.float32)
            w = jnp.exp(jnp.where(mask, s * 0.125, -30.0))
            w = w / jnp.sum(w, axis=1, keepdims=True)
            cbh = lax.dot_general(w.astype(jnp.bfloat16), vbh,
                                  (((1,), (0,)), ((), ())),
                                  preferred_element_type=jnp.float32)
            cparts.append(cbh.astype(jnp.bfloat16))
        cb = jnp.concatenate(cparts, axis=1)
        part_ref[pl.ds(b * SQ, SQ), :] = lax.dot_general(
            cb, wo, (((1,), (0,)), ((), ())),
            preferred_element_type=jnp.float32).astype(jnp.bfloat16)

        if b == 0:
            pl.semaphore_wait(barrier, N_DEV - 1)
        for o in range(1, N_DEV):
            p1_rdma(b, o).start()

    for half in range(B):
        red = part_ref[
            pl.ds(half * SQ + my * HCHUNK, HCHUNK), :].astype(jnp.float32)
        for o in range(1, N_DEV):
            p1_rdma(half, o).wait_recv()
            red = red + recv1_ref[
                half * (N_DEV - 1) + o - 1].astype(jnp.float32)
        out_ref[pl.ds(half * SQ + my * HCHUNK, HCHUNK), :] = (
            red.astype(jnp.bfloat16))
        for o in range(1, N_DEV):
            p2_rdma(half, o).start()

    for half in range(B):
        for o in range(1, N_DEV):
            p2_rdma(half, o).wait_recv()

    for half in range(B):
        for o in range(1, N_DEV):
            p1_rdma(half, o).wait_send()
            p2_rdma(half, o).wait_send()


def kernel(x, Wq, K_ext, V_ext, Wo):
    my = lax.axis_index("i")
    x2 = x.reshape(ROWS, DMODEL).astype(jnp.bfloat16)
    ks = lax.dynamic_slice_in_dim(K_ext, my * HQ_LOCAL, HQ_LOCAL, axis=2)
    vs = lax.dynamic_slice_in_dim(V_ext, my * HQ_LOCAL, HQ_LOCAL, axis=2)
    ks = jnp.transpose(ks, (0, 2, 1, 3)).astype(jnp.bfloat16)
    vs = jnp.transpose(vs, (0, 2, 1, 3)).astype(jnp.bfloat16)
    wq = Wq.astype(jnp.bfloat16)
    wo = Wo.astype(jnp.bfloat16)

    nslots = B * (N_DEV - 1)
    out = pl.pallas_call(
        _body,
        out_shape=jax.ShapeDtypeStruct((ROWS, DMODEL), jnp.bfloat16),
        in_specs=[pl.BlockSpec(memory_space=pltpu.VMEM)] * 5,
        out_specs=pl.BlockSpec(memory_space=pltpu.VMEM),
        scratch_shapes=[
            pltpu.VMEM((ROWS, DMODEL), jnp.bfloat16),
            pltpu.VMEM((nslots, HCHUNK, DMODEL), jnp.bfloat16),
            pltpu.SemaphoreType.DMA((nslots,)),
            pltpu.SemaphoreType.DMA((nslots,)),
            pltpu.SemaphoreType.DMA((nslots,)),
            pltpu.SemaphoreType.DMA((nslots,)),
        ],
        compiler_params=pltpu.CompilerParams(collective_id=0),
    )(x2, wq, ks, vs, wo)
    return out.reshape(B, SQ, DMODEL)


# device time: 24866 ns/iter; 1.0391x vs baseline; 1.0391x over previous
import jax
import jax.numpy as jnp
from jax import lax
from jax.experimental import pallas as pl
from jax.experimental.pallas import tpu as pltpu

N_DEV = 16
B, SQ, DMODEL = 2, 256, 512
HQ_LOCAL, DH = 4, 64
ROWS = B * SQ
HCHUNK = SQ // N_DEV
WINDOW = 128


def _body(x_ref, wq_ref, k_ref, v_ref, wo_ref, out_ref,
          part_ref, recv1_ref,
          send1, recv1, send2, recv2):
    my = lax.axis_index("i")

    barrier = pltpu.get_barrier_semaphore()
    for o in range(1, N_DEV):
        pl.semaphore_signal(
            barrier, inc=1,
            device_id=((my + o) % N_DEV,),
            device_id_type=pl.DeviceIdType.MESH,
        )

    qi = lax.broadcasted_iota(jnp.int32, (SQ, SQ), 0)
    ki = lax.broadcasted_iota(jnp.int32, (SQ, SQ), 1)
    mask = jnp.abs(qi - ki) <= WINDOW

    def p1_rdma(half, o):
        dst = (my + o) % N_DEV
        return pltpu.make_async_remote_copy(
            src_ref=part_ref.at[pl.ds(half * SQ + dst * HCHUNK, HCHUNK), :],
            dst_ref=recv1_ref.at[half * (N_DEV - 1) + o - 1],
            send_sem=send1.at[half * (N_DEV - 1) + o - 1],
            recv_sem=recv1.at[half * (N_DEV - 1) + o - 1],
            device_id=(dst,),
            device_id_type=pl.DeviceIdType.MESH,
        )

    def p2_rdma(half, o):
        dst = (my + o) % N_DEV
        return pltpu.make_async_remote_copy(
            src_ref=out_ref.at[pl.ds(half * SQ + my * HCHUNK, HCHUNK), :],
            dst_ref=out_ref.at[pl.ds(half * SQ + my * HCHUNK, HCHUNK), :],
            send_sem=send2.at[half * (N_DEV - 1) + o - 1],
            recv_sem=recv2.at[half * (N_DEV - 1) + o - 1],
            device_id=(dst,),
            device_id_type=pl.DeviceIdType.MESH,
        )

    wq = wq_ref[:, :].astype(jnp.bfloat16)
    wo = wo_ref[:, :].astype(jnp.bfloat16)
    for b in range(B):
        xb = x_ref[pl.ds(b * SQ, SQ), :].astype(jnp.bfloat16)
        qb = lax.dot_general(xb, wq, (((1,), (0,)), ((), ())),
                             preferred_element_type=jnp.float32)
        qb = qb.astype(jnp.bfloat16)
        cparts = []
        for h in range(HQ_LOCAL):
            qbh = qb[:, h * DH:(h + 1) * DH]
            kbh = k_ref[b, h].astype(jnp.bfloat16)
            vbh = v_ref[b, h].astype(jnp.bfloat16)
            s = lax.dot_general(qbh, kbh, (((1,), (1,)), ((), ())),
                                preferred_element_type=jnp.float32)
            w = jnp.exp(jnp.where(mask, s * 0.125, -30.0))
            w = w / jnp.sum(w, axis=1, keepdims=True)
            cbh = lax.dot_general(w.astype(jnp.bfloat16), vbh,
                                  (((1,), (0,)), ((), ())),
                                  preferred_element_type=jnp.float32)
            cparts.append(cbh.astype(jnp.bfloat16))
        cb = jnp.concatenate(cparts, axis=1)
        part_ref[pl.ds(b * SQ, SQ), :] = lax.dot_general(
            cb, wo, (((1,), (0,)), ((), ())),
            preferred_element_type=jnp.float32).astype(jnp.bfloat16)

        if b == 0:
            pl.semaphore_wait(barrier, N_DEV - 1)
        for o in range(1, N_DEV):
            p1_rdma(b, o).start()

    for half in range(B):
        red = part_ref[
            pl.ds(half * SQ + my * HCHUNK, HCHUNK), :].astype(jnp.float32)
        for o in range(1, N_DEV):
            p1_rdma(half, o).wait_recv()
            red = red + recv1_ref[
                half * (N_DEV - 1) + o - 1].astype(jnp.float32)
        out_ref[pl.ds(half * SQ + my * HCHUNK, HCHUNK), :] = (
            red.astype(jnp.bfloat16))
        for o in range(1, N_DEV):
            p2_rdma(half, o).start()

    for half in range(B):
        for o in range(1, N_DEV):
            p2_rdma(half, o).wait_recv()

    for half in range(B):
        for o in range(1, N_DEV):
            p1_rdma(half, o).wait_send()
            p2_rdma(half, o).wait_send()


def kernel(x, Wq, K_ext, V_ext, Wo):
    my = lax.axis_index("i")
    x2 = x.reshape(ROWS, DMODEL)
    ks = lax.dynamic_slice_in_dim(K_ext, my * HQ_LOCAL, HQ_LOCAL, axis=2)
    vs = lax.dynamic_slice_in_dim(V_ext, my * HQ_LOCAL, HQ_LOCAL, axis=2)
    ks = jnp.transpose(ks, (0, 2, 1, 3))
    vs = jnp.transpose(vs, (0, 2, 1, 3))

    nslots = B * (N_DEV - 1)
    out = pl.pallas_call(
        _body,
        out_shape=jax.ShapeDtypeStruct((ROWS, DMODEL), jnp.bfloat16),
        in_specs=[pl.BlockSpec(memory_space=pltpu.VMEM)] * 5,
        out_specs=pl.BlockSpec(memory_space=pltpu.VMEM),
        scratch_shapes=[
            pltpu.VMEM((ROWS, DMODEL), jnp.bfloat16),
            pltpu.VMEM((nslots, HCHUNK, DMODEL), jnp.bfloat16),
            pltpu.SemaphoreType.DMA((nslots,)),
            pltpu.SemaphoreType.DMA((nslots,)),
            pltpu.SemaphoreType.DMA((nslots,)),
            pltpu.SemaphoreType.DMA((nslots,)),
        ],
        compiler_params=pltpu.CompilerParams(collective_id=0),
    )(x2, Wq, ks, vs, Wo)
    return out.reshape(B, SQ, DMODEL)
